# baseline (speedup 1.0000x reference)
import functools

import jax
import jax.numpy as jnp
from jax import lax
from jax.experimental import pallas as pl
from jax.experimental.pallas import tpu as pltpu
from jax.experimental.pallas import tpu_sc as plsc

INPUT_DIM = 1000000
OUTPUT_DIM = 32
BATCH = 16384
SEQ = 50
SEQ_PAD = 52

NC = 2
NS = 16
NW = NC * NS

BPW = BATCH // NW
CHUNK = 128
BCH = BPW // CHUNK
CPW = SEQ * BCH
NBUF = 8
LEAD = 4
NROUND = CPW // NBUF

T1_COLS = 2048
T1_GRID = (INPUT_DIM + T1_COLS - 1) // T1_COLS
TBL_ROWS = T1_GRID * T1_COLS
T2_BATCH = 512
FPB = SEQ_PAD * OUTPUT_DIM


def _t1_body(x_ref, o_ref):
  x = x_ref[...]
  eye = jnp.eye(OUTPUT_DIM, dtype=jnp.float32)
  y = lax.dot_general(x, eye, (((0,), (0,)), ((), ())),
                      preferred_element_type=jnp.float32)
  q = T1_COLS // 4
  o_ref[...] = jnp.concatenate(
      [y[k * q:(k + 1) * q, :] for k in range(4)], axis=1)


def _table_linearize(table_t):
  return pl.pallas_call(
      _t1_body,
      grid=(T1_GRID,),
      in_specs=[pl.BlockSpec((OUTPUT_DIM, T1_COLS), lambda i: (0, i))],
      out_specs=pl.BlockSpec((T1_COLS // 4, 128), lambda i: (i, 0)),
      out_shape=jax.ShapeDtypeStruct((TBL_ROWS // 4, 128), jnp.float32),
  )(table_t)


def _t2_body(x_ref, o_ref):
  o_ref[...] = x_ref[...].T


def _out_transpose(out_flat):
  return pl.pallas_call(
      _t2_body,
      grid=(BATCH // T2_BATCH,),
      in_specs=[pl.BlockSpec((T2_BATCH, FPB), lambda i: (i, 0))],
      out_specs=pl.BlockSpec((FPB, T2_BATCH), lambda i: (0, i)),
      out_shape=jax.ShapeDtypeStruct((FPB, BATCH), jnp.float32),
  )(out_flat)


def _make_sc_kernel():
  mesh = plsc.VectorSubcoreMesh(core_axis_name="c", subcore_axis_name="s")

  @functools.partial(
      pl.kernel,
      out_type=jax.ShapeDtypeStruct((BATCH, FPB), jnp.float32),
      mesh=mesh,
      compiler_params=pltpu.CompilerParams(use_tc_tiling_on_sc=False),
      scratch_types=[
          pltpu.VMEM((SEQ, BPW), jnp.int32),
          [pltpu.VMEM((CHUNK, OUTPUT_DIM), jnp.float32)] * NBUF,
          [pltpu.SemaphoreType.DMA] * NBUF,
          [pltpu.SemaphoreType.DMA] * NBUF,
      ],
  )
  def k(idxt_hbm, table_hbm, out_hbm, idx_v, bufs, gsems, wsems):
    wid = lax.axis_index("s") * NC + lax.axis_index("c")
    b0 = wid * BPW
    pltpu.sync_copy(idxt_hbm.at[:, pl.ds(b0, BPW)], idx_v)

    def gather(j, b):
      s = j // BCH
      bc = (j % BCH) * CHUNK
      pltpu.async_copy(
          table_hbm.at[idx_v.at[s, pl.ds(bc, CHUNK)]], bufs[b], gsems[b])

    def gather_wait(b):
      pltpu.make_async_copy(
          table_hbm.at[idx_v.at[0, pl.ds(0, CHUNK)]], bufs[b],
          gsems[b]).wait()

    def write(j, b):
      s = j // BCH
      bc = (j % BCH) * CHUNK
      pltpu.async_copy(
          bufs[b],
          out_hbm.at[pl.ds(b0 + bc, CHUNK), pl.ds(s * OUTPUT_DIM,
                                                  OUTPUT_DIM)],
          wsems[b])

    def write_wait(b):
      pltpu.make_async_copy(
          bufs[b], out_hbm.at[pl.ds(0, CHUNK), pl.ds(0, OUTPUT_DIM)],
          wsems[b]).wait()

    for b in range(LEAD):
      gather(b, b)

    def round_body(g, _):
      for b in range(NBUF):
        j = g * NBUF + b
        gather_wait(b)
        write(j, b)
        bb = (b + LEAD) % NBUF
        jn = j + LEAD
        if b < NBUF - LEAD:
          @pl.when(g >= 1)
          def _():
            write_wait(bb)
          gather(jn, bb)
        else:
          write_wait(bb)
          @pl.when(g < NROUND - 1)
          def _():
            gather(jn, bb)
      return _

    lax.fori_loop(0, NROUND, round_body, None)

    for b in range(NBUF - LEAD, NBUF):
      write_wait(b)

  return k


_sc_gather = _make_sc_kernel()


@jax.jit
def kernel(inputs, embeddings):
  table_lin = _table_linearize(embeddings.T)
  tbl = table_lin.reshape(TBL_ROWS, OUTPUT_DIM)
  t = inputs.astype(jnp.int32)
  idx_s = ((t >> 11) << 11) | ((t & 511) << 2) | ((t >> 9) & 3)
  out = _sc_gather(idx_s.T, tbl)
  out_t = _out_transpose(out)
  return (out_t[:SEQ * OUTPUT_DIM]
          .reshape(SEQ, OUTPUT_DIM, BATCH).transpose(2, 0, 1))


# speedup vs baseline: 1.2801x; 1.2801x over previous
import functools

import jax
import jax.numpy as jnp
from jax import lax
from jax.experimental import pallas as pl
from jax.experimental.pallas import tpu as pltpu
from jax.experimental.pallas import tpu_sc as plsc

INPUT_DIM = 1000000
OUTPUT_DIM = 32
BATCH = 16384
SEQ = 50
SEQ_PAD = 52

NC = 2
NS = 16
NW = NC * NS

BPW = BATCH // NW
CHUNK = 128
BCH = BPW // CHUNK
CPW = SEQ * BCH
NBUF = 8
LEAD = 4
NROUND = CPW // NBUF

T1_COLS = 8192
T1_GRID = (INPUT_DIM + T1_COLS - 1) // T1_COLS
TBL_ROWS = T1_GRID * T1_COLS
T2_BATCH = 512
FPB = SEQ_PAD * OUTPUT_DIM


def _t1_body(x_ref, o_ref):
  y = x_ref[...].T
  q = T1_COLS // 4
  o_ref[...] = jnp.concatenate(
      [y[k * q:(k + 1) * q, :] for k in range(4)], axis=1)


def _table_linearize(table_t):
  return pl.pallas_call(
      _t1_body,
      grid=(T1_GRID,),
      in_specs=[pl.BlockSpec((OUTPUT_DIM, T1_COLS), lambda i: (0, i))],
      out_specs=pl.BlockSpec((T1_COLS // 4, 128), lambda i: (i, 0)),
      out_shape=jax.ShapeDtypeStruct((TBL_ROWS // 4, 128), jnp.float32),
  )(table_t)


def _t2_body(x_ref, o_ref):
  o_ref[...] = x_ref[...].T


def _out_transpose(out_flat):
  return pl.pallas_call(
      _t2_body,
      grid=(BATCH // T2_BATCH,),
      in_specs=[pl.BlockSpec((T2_BATCH, FPB), lambda i: (i, 0))],
      out_specs=pl.BlockSpec((FPB, T2_BATCH), lambda i: (0, i)),
      out_shape=jax.ShapeDtypeStruct((FPB, BATCH), jnp.float32),
  )(out_flat)


def _make_sc_kernel():
  mesh = plsc.VectorSubcoreMesh(core_axis_name="c", subcore_axis_name="s")

  @functools.partial(
      pl.kernel,
      out_type=jax.ShapeDtypeStruct((BATCH, FPB), jnp.float32),
      mesh=mesh,
      compiler_params=pltpu.CompilerParams(use_tc_tiling_on_sc=False),
      scratch_types=[
          pltpu.VMEM((SEQ, BPW), jnp.int32),
          [pltpu.VMEM((CHUNK, OUTPUT_DIM), jnp.float32)] * NBUF,
          [pltpu.SemaphoreType.DMA] * NBUF,
          [pltpu.SemaphoreType.DMA] * NBUF,
      ],
  )
  def k(idxt_hbm, table_hbm, out_hbm, idx_v, bufs, gsems, wsems):
    wid = lax.axis_index("s") * NC + lax.axis_index("c")
    b0 = wid * BPW
    pltpu.sync_copy(idxt_hbm.at[:, pl.ds(b0, BPW)], idx_v)

    def gather(j, b):
      s = j // BCH
      bc = (j % BCH) * CHUNK
      pltpu.async_copy(
          table_hbm.at[idx_v.at[s, pl.ds(bc, CHUNK)]], bufs[b], gsems[b])

    def gather_wait(b):
      pltpu.make_async_copy(
          table_hbm.at[idx_v.at[0, pl.ds(0, CHUNK)]], bufs[b],
          gsems[b]).wait()

    def write(j, b):
      s = j // BCH
      bc = (j % BCH) * CHUNK
      pltpu.async_copy(
          bufs[b],
          out_hbm.at[pl.ds(b0 + bc, CHUNK), pl.ds(s * OUTPUT_DIM,
                                                  OUTPUT_DIM)],
          wsems[b])

    def write_wait(b):
      pltpu.make_async_copy(
          bufs[b], out_hbm.at[pl.ds(0, CHUNK), pl.ds(0, OUTPUT_DIM)],
          wsems[b]).wait()

    for b in range(LEAD):
      gather(b, b)

    def round_body(g, _):
      for b in range(NBUF):
        j = g * NBUF + b
        gather_wait(b)
        write(j, b)
        bb = (b + LEAD) % NBUF
        jn = j + LEAD
        if b < NBUF - LEAD:
          @pl.when(g >= 1)
          def _():
            write_wait(bb)
          gather(jn, bb)
        else:
          write_wait(bb)
          @pl.when(g < NROUND - 1)
          def _():
            gather(jn, bb)
      return _

    lax.fori_loop(0, NROUND, round_body, None)

    for b in range(NBUF - LEAD, NBUF):
      write_wait(b)

  return k


_sc_gather = _make_sc_kernel()


@jax.jit
def kernel(inputs, embeddings):
  table_lin = _table_linearize(embeddings.T)
  tbl = table_lin.reshape(TBL_ROWS, OUTPUT_DIM)
  t = inputs.astype(jnp.int32)
  lb = T1_COLS.bit_length() - 1
  qm = T1_COLS // 4 - 1
  idx_s = ((t >> lb) << lb) | ((t & qm) << 2) | ((t >> (lb - 2)) & 3)
  out = _sc_gather(idx_s.T, tbl)
  out_t = _out_transpose(out)
  return (out_t[:SEQ * OUTPUT_DIM]
          .reshape(SEQ, OUTPUT_DIM, BATCH).transpose(2, 0, 1))


# speedup vs baseline: 1.3798x; 1.0779x over previous
import functools

import jax
import jax.numpy as jnp
from jax import lax
from jax.experimental import pallas as pl
from jax.experimental.pallas import tpu as pltpu
from jax.experimental.pallas import tpu_sc as plsc

INPUT_DIM = 1000000
OUTPUT_DIM = 32
BATCH = 16384
SEQ = 50
SEQ_PAD = 52

NC = 2
NS = 16
NW = NC * NS

BPW = BATCH // NW
CHUNK = 128
BCH = BPW // CHUNK
CPW = SEQ * BCH
NBUF = 8
LEAD = 4
NROUND = CPW // NBUF

T1_COLS = 16384
T1_GRID = (INPUT_DIM + T1_COLS - 1) // T1_COLS
TBL_ROWS = T1_GRID * T1_COLS
T2_BATCH = 512
FPB = SEQ_PAD * OUTPUT_DIM


def _t1_body(x_ref, o_ref):
  y = x_ref[...].T
  q = T1_COLS // 4
  o_ref[...] = jnp.concatenate(
      [y[k * q:(k + 1) * q, :] for k in range(4)], axis=1)


def _table_linearize(table_t):
  return pl.pallas_call(
      _t1_body,
      grid=(T1_GRID,),
      in_specs=[pl.BlockSpec((OUTPUT_DIM, T1_COLS), lambda i: (0, i))],
      out_specs=pl.BlockSpec((T1_COLS // 4, 128), lambda i: (i, 0)),
      out_shape=jax.ShapeDtypeStruct((TBL_ROWS // 4, 128), jnp.float32),
  )(table_t)


NTILE = FPB // 128
FOUT = SEQ * OUTPUT_DIM


def _t2_body(x_ref, o_ref):
  x = x_ref[...]
  for j in range(NTILE):
    y = x[:, j, :].T
    if 128 * (j + 1) <= FOUT:
      o_ref[128 * j:128 * (j + 1), :] = y
    else:
      o_ref[128 * j:FOUT, :] = y[:FOUT - 128 * j, :]


def _out_transpose(out3d):
  return pl.pallas_call(
      _t2_body,
      grid=(BATCH // T2_BATCH,),
      in_specs=[pl.BlockSpec((T2_BATCH, NTILE, 128), lambda i: (i, 0, 0))],
      out_specs=pl.BlockSpec((FOUT, T2_BATCH), lambda i: (0, i)),
      out_shape=jax.ShapeDtypeStruct((FOUT, BATCH), jnp.float32),
  )(out3d)


def _make_sc_kernel():
  mesh = plsc.VectorSubcoreMesh(core_axis_name="c", subcore_axis_name="s")

  @functools.partial(
      pl.kernel,
      out_type=jax.ShapeDtypeStruct((BATCH, FPB // 128, 128), jnp.float32),
      mesh=mesh,
      compiler_params=pltpu.CompilerParams(use_tc_tiling_on_sc=False),
      scratch_types=[
          pltpu.VMEM((SEQ, BPW), jnp.int32),
          [pltpu.VMEM((CHUNK, OUTPUT_DIM), jnp.float32)] * NBUF,
          [pltpu.SemaphoreType.DMA] * NBUF,
          [pltpu.SemaphoreType.DMA] * NBUF,
      ],
  )
  def k(idxt_hbm, table_hbm, out_hbm, idx_v, bufs, gsems, wsems):
    wid = lax.axis_index("s") * NC + lax.axis_index("c")
    b0 = wid * BPW
    pltpu.sync_copy(idxt_hbm.at[:, pl.ds(b0, BPW)], idx_v)

    def gather(j, b):
      s = j // BCH
      bc = (j % BCH) * CHUNK
      pltpu.async_copy(
          table_hbm.at[idx_v.at[s, pl.ds(bc, CHUNK)]], bufs[b], gsems[b])

    def gather_wait(b):
      pltpu.make_async_copy(
          table_hbm.at[idx_v.at[0, pl.ds(0, CHUNK)]], bufs[b],
          gsems[b]).wait()

    def write(j, b):
      s = j // BCH
      bc = (j % BCH) * CHUNK
      pltpu.async_copy(
          bufs[b],
          out_hbm.at[pl.ds(b0 + bc, CHUNK), s // 4,
                     pl.ds((s % 4) * OUTPUT_DIM, OUTPUT_DIM)],
          wsems[b])

    def write_wait(b):
      pltpu.make_async_copy(
          bufs[b], out_hbm.at[pl.ds(0, CHUNK), 0, pl.ds(0, OUTPUT_DIM)],
          wsems[b]).wait()

    for b in range(LEAD):
      gather(b, b)

    def round_body(g, _):
      for b in range(NBUF):
        j = g * NBUF + b
        gather_wait(b)
        write(j, b)
        bb = (b + LEAD) % NBUF
        jn = j + LEAD
        if b < NBUF - LEAD:
          @pl.when(g >= 1)
          def _():
            write_wait(bb)
          gather(jn, bb)
        else:
          write_wait(bb)
          @pl.when(g < NROUND - 1)
          def _():
            gather(jn, bb)
      return _

    lax.fori_loop(0, NROUND, round_body, None)

    for b in range(NBUF - LEAD, NBUF):
      write_wait(b)

  return k


_sc_gather = _make_sc_kernel()


@jax.jit
def kernel(inputs, embeddings):
  table_lin = _table_linearize(embeddings.T)
  tbl = table_lin.reshape(TBL_ROWS, OUTPUT_DIM)
  t = inputs.astype(jnp.int32)
  lb = T1_COLS.bit_length() - 1
  qm = T1_COLS // 4 - 1
  idx_s = ((t >> lb) << lb) | ((t & qm) << 2) | ((t >> (lb - 2)) & 3)
  out = _sc_gather(idx_s.T, tbl)
  out_t = _out_transpose(out)
  return out_t.reshape(SEQ, OUTPUT_DIM, BATCH).transpose(2, 0, 1)


# speedup vs baseline: 1.8365x; 1.3310x over previous
import functools

import jax
import jax.numpy as jnp
from jax import lax
from jax.experimental import pallas as pl
from jax.experimental.pallas import tpu as pltpu
from jax.experimental.pallas import tpu_sc as plsc

INPUT_DIM = 1000000
OUTPUT_DIM = 32
BATCH = 16384
SEQ = 50
SEQ_PAD = 52

NC = 2
NS = 16
NW = NC * NS

BPW = BATCH // NW
CHUNK = 128
BCH = BPW // CHUNK
CPW = SEQ * BCH
NBUF = 8
LEAD = 4
NROUND = CPW // NBUF

T1_COLS = 16384
T1_GRID = (INPUT_DIM + T1_COLS - 1) // T1_COLS
TBL_ROWS = T1_GRID * T1_COLS
T2_BATCH = 512
FPB = SEQ_PAD * OUTPUT_DIM


def _t1_body(x_ref, o_ref):
  y = x_ref[...].T
  q = T1_COLS // 4
  o_ref[...] = jnp.concatenate(
      [y[k * q:(k + 1) * q, :] for k in range(4)], axis=1)


def _table_linearize(table_t):
  return pl.pallas_call(
      _t1_body,
      grid=(T1_GRID,),
      in_specs=[pl.BlockSpec((OUTPUT_DIM, T1_COLS), lambda i: (0, i))],
      out_specs=pl.BlockSpec((T1_COLS // 4, 128), lambda i: (i, 0)),
      out_shape=jax.ShapeDtypeStruct((TBL_ROWS // 4, 128), jnp.float32),
  )(table_t)


NTILE = FPB // 128
FOUT = SEQ * OUTPUT_DIM


def _t2_body(x_ref, o_ref):
  x = x_ref[...]
  for j in range(NTILE):
    y = x[j].T
    if 128 * (j + 1) <= FOUT:
      o_ref[128 * j:128 * (j + 1), :] = y
    else:
      o_ref[128 * j:FOUT, :] = y[:FOUT - 128 * j, :]


def _out_transpose(out3d):
  return pl.pallas_call(
      _t2_body,
      grid=(BATCH // T2_BATCH,),
      in_specs=[pl.BlockSpec((NTILE, T2_BATCH, 128), lambda i: (0, i, 0))],
      out_specs=pl.BlockSpec((FOUT, T2_BATCH), lambda i: (0, i)),
      out_shape=jax.ShapeDtypeStruct((FOUT, BATCH), jnp.float32),
  )(out3d)


def _make_sc_kernel():
  mesh = plsc.VectorSubcoreMesh(core_axis_name="c", subcore_axis_name="s")

  @functools.partial(
      pl.kernel,
      out_type=jax.ShapeDtypeStruct((FPB // 128, BATCH, 128), jnp.float32),
      mesh=mesh,
      compiler_params=pltpu.CompilerParams(use_tc_tiling_on_sc=False),
      scratch_types=[
          pltpu.VMEM((SEQ, BPW), jnp.int32),
          [pltpu.VMEM((CHUNK, OUTPUT_DIM), jnp.float32)] * NBUF,
          [pltpu.SemaphoreType.DMA] * NBUF,
          [pltpu.SemaphoreType.DMA] * NBUF,
      ],
  )
  def k(idxt_hbm, table_hbm, out_hbm, idx_v, bufs, gsems, wsems):
    wid = lax.axis_index("s") * NC + lax.axis_index("c")
    b0 = wid * BPW
    pltpu.sync_copy(idxt_hbm.at[:, pl.ds(b0, BPW)], idx_v)

    def gather(j, b):
      s = j // BCH
      bc = (j % BCH) * CHUNK
      pltpu.async_copy(
          table_hbm.at[idx_v.at[s, pl.ds(bc, CHUNK)]], bufs[b], gsems[b])

    def gather_wait(b):
      pltpu.make_async_copy(
          table_hbm.at[idx_v.at[0, pl.ds(0, CHUNK)]], bufs[b],
          gsems[b]).wait()

    def write(j, b):
      s = j // BCH
      bc = (j % BCH) * CHUNK
      pltpu.async_copy(
          bufs[b],
          out_hbm.at[s // 4, pl.ds(b0 + bc, CHUNK),
                     pl.ds((s % 4) * OUTPUT_DIM, OUTPUT_DIM)],
          wsems[b])

    def write_wait(b):
      pltpu.make_async_copy(
          bufs[b], out_hbm.at[0, pl.ds(0, CHUNK), pl.ds(0, OUTPUT_DIM)],
          wsems[b]).wait()

    for b in range(LEAD):
      gather(b, b)

    def round_body(g, _):
      for b in range(NBUF):
        j = g * NBUF + b
        gather_wait(b)
        write(j, b)
        bb = (b + LEAD) % NBUF
        jn = j + LEAD
        if b < NBUF - LEAD:
          @pl.when(g >= 1)
          def _():
            write_wait(bb)
          gather(jn, bb)
        else:
          write_wait(bb)
          @pl.when(g < NROUND - 1)
          def _():
            gather(jn, bb)
      return _

    lax.fori_loop(0, NROUND, round_body, None)

    for b in range(NBUF - LEAD, NBUF):
      write_wait(b)

  return k


_sc_gather = _make_sc_kernel()


@jax.jit
def kernel(inputs, embeddings):
  table_lin = _table_linearize(embeddings.T)
  tbl = table_lin.reshape(TBL_ROWS, OUTPUT_DIM)
  t = inputs.astype(jnp.int32)
  lb = T1_COLS.bit_length() - 1
  qm = T1_COLS // 4 - 1
  idx_s = ((t >> lb) << lb) | ((t & qm) << 2) | ((t >> (lb - 2)) & 3)
  out = _sc_gather(idx_s.T, tbl)
  out_t = _out_transpose(out)
  return out_t.reshape(SEQ, OUTPUT_DIM, BATCH).transpose(2, 0, 1)


# speedup vs baseline: 1.8456x; 1.0050x over previous
import functools

import jax
import jax.numpy as jnp
from jax import lax
from jax.experimental import pallas as pl
from jax.experimental.pallas import tpu as pltpu
from jax.experimental.pallas import tpu_sc as plsc

INPUT_DIM = 1000000
OUTPUT_DIM = 32
BATCH = 16384
SEQ = 50
SEQ_PAD = 52

NC = 2
NS = 16
NW = NC * NS

BPW = BATCH // NW
CHUNK = 128
BCH = BPW // CHUNK
CPW = SEQ * BCH
NBUF = 8
LEAD = 4
NROUND = CPW // NBUF

T1_COLS = 32768
T1_GRID = (INPUT_DIM + T1_COLS - 1) // T1_COLS
TBL_ROWS = T1_GRID * T1_COLS
T2_BATCH = 512
FPB = SEQ_PAD * OUTPUT_DIM


def _t1_body(x_ref, o_ref):
  y = x_ref[...].T
  q = T1_COLS // 4
  o_ref[...] = jnp.concatenate(
      [y[k * q:(k + 1) * q, :] for k in range(4)], axis=1)


def _table_linearize(table_t):
  return pl.pallas_call(
      _t1_body,
      grid=(T1_GRID,),
      in_specs=[pl.BlockSpec((OUTPUT_DIM, T1_COLS), lambda i: (0, i))],
      out_specs=pl.BlockSpec((T1_COLS // 4, 128), lambda i: (i, 0)),
      out_shape=jax.ShapeDtypeStruct((TBL_ROWS // 4, 128), jnp.float32),
  )(table_t)


NTILE = FPB // 128
FOUT = SEQ * OUTPUT_DIM


def _t2_body(x_ref, o_ref):
  x = x_ref[...]
  for j in range(NTILE):
    y = x[j].T
    if 128 * (j + 1) <= FOUT:
      o_ref[128 * j:128 * (j + 1), :] = y
    else:
      o_ref[128 * j:FOUT, :] = y[:FOUT - 128 * j, :]


def _out_transpose(out3d):
  return pl.pallas_call(
      _t2_body,
      grid=(BATCH // T2_BATCH,),
      in_specs=[pl.BlockSpec((NTILE, T2_BATCH, 128), lambda i: (0, i, 0))],
      out_specs=pl.BlockSpec((FOUT, T2_BATCH), lambda i: (0, i)),
      out_shape=jax.ShapeDtypeStruct((FOUT, BATCH), jnp.float32),
  )(out3d)


def _make_sc_kernel():
  mesh = plsc.VectorSubcoreMesh(core_axis_name="c", subcore_axis_name="s")

  @functools.partial(
      pl.kernel,
      out_type=jax.ShapeDtypeStruct((FPB // 128, BATCH, 128), jnp.float32),
      mesh=mesh,
      compiler_params=pltpu.CompilerParams(use_tc_tiling_on_sc=False),
      scratch_types=[
          pltpu.VMEM((SEQ, BPW), jnp.int32),
          [pltpu.VMEM((CHUNK, OUTPUT_DIM), jnp.float32)] * NBUF,
          [pltpu.SemaphoreType.DMA] * NBUF,
          [pltpu.SemaphoreType.DMA] * NBUF,
      ],
  )
  def k(idxt_hbm, table_hbm, out_hbm, idx_v, bufs, gsems, wsems):
    wid = lax.axis_index("s") * NC + lax.axis_index("c")
    b0 = wid * BPW
    pltpu.sync_copy(idxt_hbm.at[:, pl.ds(b0, BPW)], idx_v)

    def gather(j, b):
      s = j // BCH
      bc = (j % BCH) * CHUNK
      pltpu.async_copy(
          table_hbm.at[idx_v.at[s, pl.ds(bc, CHUNK)]], bufs[b], gsems[b])

    def gather_wait(b):
      pltpu.make_async_copy(
          table_hbm.at[idx_v.at[0, pl.ds(0, CHUNK)]], bufs[b],
          gsems[b]).wait()

    def write(j, b):
      s = j // BCH
      bc = (j % BCH) * CHUNK
      pltpu.async_copy(
          bufs[b],
          out_hbm.at[s // 4, pl.ds(b0 + bc, CHUNK),
                     pl.ds((s % 4) * OUTPUT_DIM, OUTPUT_DIM)],
          wsems[b])

    def write_wait(b):
      pltpu.make_async_copy(
          bufs[b], out_hbm.at[0, pl.ds(0, CHUNK), pl.ds(0, OUTPUT_DIM)],
          wsems[b]).wait()

    for b in range(LEAD):
      gather(b, b)

    def round_body(g, _):
      for b in range(NBUF):
        j = g * NBUF + b
        gather_wait(b)
        write(j, b)
        bb = (b + LEAD) % NBUF
        jn = j + LEAD
        if b < NBUF - LEAD:
          @pl.when(g >= 1)
          def _():
            write_wait(bb)
          gather(jn, bb)
        else:
          write_wait(bb)
          @pl.when(g < NROUND - 1)
          def _():
            gather(jn, bb)
      return _

    lax.fori_loop(0, NROUND, round_body, None)

    for b in range(NBUF - LEAD, NBUF):
      write_wait(b)

  return k


_sc_gather = _make_sc_kernel()


@jax.jit
def kernel(inputs, embeddings):
  table_lin = _table_linearize(embeddings.T)
  tbl = table_lin.reshape(TBL_ROWS, OUTPUT_DIM)
  t = inputs.astype(jnp.int32)
  lb = T1_COLS.bit_length() - 1
  qm = T1_COLS // 4 - 1
  idx_s = ((t >> lb) << lb) | ((t & qm) << 2) | ((t >> (lb - 2)) & 3)
  out = _sc_gather(idx_s.T, tbl)
  out_t = _out_transpose(out)
  return out_t.reshape(SEQ, OUTPUT_DIM, BATCH).transpose(2, 0, 1)
